# baseline (speedup 1.0000x reference)
import functools

import jax
import jax.numpy as jnp
from jax import lax
from jax.experimental import pallas as pl
from jax.experimental.pallas import tpu as pltpu
from jax.experimental.pallas import tpu_sc as plsc

N = 10000
D = 128
HEADS = 8
C1 = 16
OUT = 40
E = 320000

NPAD = 10016
DUMMY = N
W1W = D + 8
W2W = 48
NW = 32
CH = 84
CHUNK = 128
NPW = CH * CHUNK
EP = NW * NPW
ROWS_PER_TILE = NPAD // 16
TCBLK = 2504


def _mesh():
    return plsc.VectorSubcoreMesh(core_axis_name="c", subcore_axis_name="s")


def _tc1_body(x_ref, w_ref, ones_ref, cab_ref, cba_ref,
              h_ref, ab_ref, ba_ref, g_ref):
    xb = x_ref[...]
    hb = jnp.dot(xb, w_ref[...], preferred_element_type=jnp.float32)
    hb = hb + ones_ref[...]
    h_ref[...] = hb
    ab = jnp.dot(hb, cab_ref[...], preferred_element_type=jnp.float32)
    ba = jnp.dot(hb, cba_ref[...], preferred_element_type=jnp.float32)
    ab_ref[...] = ab
    ba_ref[...] = ba
    m = jnp.max(ab, axis=0, keepdims=True)
    i = pl.program_id(0)

    @pl.when(i == 0)
    def _():
        g_ref[...] = m

    @pl.when(i > 0)
    def _():
        g_ref[...] = jnp.maximum(g_ref[...], m)


def _tc1(xpad, W1e, ones1, cab, cba):
    grid = NPAD // TCBLK
    return pl.pallas_call(
        _tc1_body,
        grid=(grid,),
        in_specs=[
            pl.BlockSpec((TCBLK, D), lambda i: (i, 0)),
            pl.BlockSpec((D, W1W), lambda i: (0, 0)),
            pl.BlockSpec((1, W1W), lambda i: (0, 0)),
            pl.BlockSpec((W1W, 16), lambda i: (0, 0)),
            pl.BlockSpec((W1W, 16), lambda i: (0, 0)),
        ],
        out_specs=[
            pl.BlockSpec((TCBLK, W1W), lambda i: (i, 0)),
            pl.BlockSpec((TCBLK, 16), lambda i: (i, 0)),
            pl.BlockSpec((TCBLK, 16), lambda i: (i, 0)),
            pl.BlockSpec((1, 16), lambda i: (0, 0)),
        ],
        out_shape=[
            jax.ShapeDtypeStruct((NPAD, W1W), jnp.float32),
            jax.ShapeDtypeStruct((NPAD, 16), jnp.float32),
            jax.ShapeDtypeStruct((NPAD, 16), jnp.float32),
            jax.ShapeDtypeStruct((1, 16), jnp.float32),
        ],
    )(xpad, W1e, ones1, cab, cba)


def _tc2_body(a_ref, b_ref, rexp_ref, bias_ref, w2_ref, ones_ref,
              t2_ref, t2b_ref, g2_ref, ab_ref, ba_ref, gm_ref):
    acc = a_ref[...] + b_ref[...]
    msg = acc[:, :D]
    den = jnp.dot(acc[:, D:], rexp_ref[...], preferred_element_type=jnp.float32)
    hb = msg / (den + 1e-16) + bias_ref[...]
    hb = jnp.where(hb > 0, hb, jnp.exp(hb) - 1.0)
    g2 = jnp.dot(hb, w2_ref[...], preferred_element_type=jnp.float32)
    g2 = g2 + ones_ref[...]
    g2_ref[...] = g2
    ab = jnp.dot(g2, t2_ref[...], preferred_element_type=jnp.float32)
    ba = jnp.dot(g2, t2b_ref[...], preferred_element_type=jnp.float32)
    ab_ref[...] = ab
    ba_ref[...] = ba
    m = jnp.max(ab, axis=0, keepdims=True)
    i = pl.program_id(0)

    @pl.when(i == 0)
    def _():
        gm_ref[...] = m

    @pl.when(i > 0)
    def _():
        gm_ref[...] = jnp.maximum(gm_ref[...], m)


def _tc2(accA, accB, rexp, b1r, W2e, ones2, t2, t2b):
    grid = NPAD // TCBLK
    return pl.pallas_call(
        _tc2_body,
        grid=(grid,),
        in_specs=[
            pl.BlockSpec((TCBLK, W1W), lambda i: (i, 0)),
            pl.BlockSpec((TCBLK, W1W), lambda i: (i, 0)),
            pl.BlockSpec((8, D), lambda i: (0, 0)),
            pl.BlockSpec((1, D), lambda i: (0, 0)),
            pl.BlockSpec((D, W2W), lambda i: (0, 0)),
            pl.BlockSpec((1, W2W), lambda i: (0, 0)),
            pl.BlockSpec((W2W, 16), lambda i: (0, 0)),
            pl.BlockSpec((W2W, 16), lambda i: (0, 0)),
        ],
        out_specs=[
            pl.BlockSpec((TCBLK, W2W), lambda i: (i, 0)),
            pl.BlockSpec((TCBLK, 16), lambda i: (i, 0)),
            pl.BlockSpec((TCBLK, 16), lambda i: (i, 0)),
            pl.BlockSpec((1, 16), lambda i: (0, 0)),
        ],
        out_shape=[
            jax.ShapeDtypeStruct((NPAD, W2W), jnp.float32),
            jax.ShapeDtypeStruct((NPAD, 16), jnp.float32),
            jax.ShapeDtypeStruct((NPAD, 16), jnp.float32),
            jax.ShapeDtypeStruct((1, 16), jnp.float32),
        ],
    )(accA, accB, rexp, b1r, W2e, ones2, t2, t2b)


def _tc3_body(a_ref, b_ref, r2_ref, bias_ref, o_ref):
    acc = a_ref[...] + b_ref[...]
    den = jnp.dot(acc[:, OUT:], r2_ref[...], preferred_element_type=jnp.float32)
    o_ref[...] = acc / (den + 1e-16) + bias_ref[...]


def _tc3(accA, accB, r2, b2p):
    grid = NPAD // TCBLK
    return pl.pallas_call(
        _tc3_body,
        grid=(grid,),
        in_specs=[
            pl.BlockSpec((TCBLK, W2W), lambda i: (i, 0)),
            pl.BlockSpec((TCBLK, W2W), lambda i: (i, 0)),
            pl.BlockSpec((8, W2W), lambda i: (0, 0)),
            pl.BlockSpec((1, W2W), lambda i: (0, 0)),
        ],
        out_specs=pl.BlockSpec((TCBLK, W2W), lambda i: (i, 0)),
        out_shape=jax.ShapeDtypeStruct((NPAD, W2W), jnp.float32),
    )(accA, accB, r2, b2p)


def _sc_edge_body(nheads, width, ab_hbm, ba_hbm, src_hbm, dst_hbm, g_hbm,
                  tab_hbm, zw_hbm, acc_hbm,
                  sidxA, didxA, raA, rbA, rowA,
                  sidxB, didxB, raB, rbB, rowB,
                  gv, acc, semA, semB):
    cid = lax.axis_index("c")
    sid = lax.axis_index("s")
    wid = sid * 2 + cid
    row0 = sid * ROWS_PER_TILE
    pltpu.sync_copy(zw_hbm.at[pl.ds(row0, ROWS_PER_TILE)],
                    acc.at[pl.ds(row0, ROWS_PER_TILE)])
    pltpu.sync_copy(g_hbm, gv)
    plsc.subcore_barrier()
    g = gv[...]
    lane = lax.iota(jnp.int32, 16)
    shift_idx = jnp.where(lane >= 8, lane - 8, 0)

    def issue(c, sidx, didx, ra, rb, row, sem):
        base = pl.multiple_of((wid * CH + jnp.minimum(c, CH - 1)) * CHUNK, CHUNK)
        pltpu.sync_copy(src_hbm.at[pl.ds(base, CHUNK)], sidx)
        pltpu.sync_copy(dst_hbm.at[pl.ds(base, CHUNK)], didx)
        cp1 = pltpu.async_copy(ab_hbm.at[sidx], ra, sem)
        cp2 = pltpu.async_copy(ba_hbm.at[didx], rb, sem)
        cp3 = pltpu.async_copy(tab_hbm.at[sidx], row, sem)
        return cp1, cp2, cp3

    def compute(didx, ra, rb, row, cps):
        for cp in cps:
            cp.wait()

        def inner(k, c2):
            e = ra[k, :] + rb[k, :]
            e = jnp.maximum(e, 0.2 * e)
            p = jnp.exp(e - g)
            if nheads == 1:
                p0 = p[0]
                for j in range(width // 16):
                    row[k, j * 16:(j + 1) * 16] = row[k, j * 16:(j + 1) * 16] * p0
            else:
                for h in range(nheads):
                    row[k, h * 16:(h + 1) * 16] = (
                        row[k, h * 16:(h + 1) * 16] * p[h])
                ps = jnp.where(lane >= 8, jnp.take(p, shift_idx), 1.0)
                row[k, width - 16:width] = row[k, width - 16:width] * ps
            return c2

        lax.fori_loop(0, CHUNK, inner, 0, unroll=4)
        pltpu.sync_copy(row, acc.at[didx], add=True)
        return None

    cpsA = issue(0, sidxA, didxA, raA, rbA, rowA, semA)
    cpsB = issue(1, sidxB, didxB, raB, rbB, rowB, semB)

    def outer(j, carry):
        compute(didxA, raA, rbA, rowA, cpsA)
        issue(2 * j + 2, sidxA, didxA, raA, rbA, rowA, semA)
        compute(didxB, raB, rbB, rowB, cpsB)
        issue(2 * j + 3, sidxB, didxB, raB, rbB, rowB, semB)
        return carry

    lax.fori_loop(0, CH // 2, outer, 0)
    for cps in (cpsA, cpsB):
        for cp in cps:
            cp.wait()
    plsc.subcore_barrier()
    pltpu.sync_copy(acc.at[pl.ds(row0, ROWS_PER_TILE)],
                    acc_hbm.at[cid].at[pl.ds(row0, ROWS_PER_TILE)])


def _sc_edge(nheads, width, ab, ba, src, dst, gvec, table, zw):
    bufs = []
    for _ in range(2):
        bufs += [
            pltpu.VMEM((CHUNK,), jnp.int32),
            pltpu.VMEM((CHUNK,), jnp.int32),
            pltpu.VMEM((CHUNK, 16), jnp.float32),
            pltpu.VMEM((CHUNK, 16), jnp.float32),
            pltpu.VMEM((CHUNK, width), jnp.float32),
        ]
    f = pl.kernel(
        functools.partial(_sc_edge_body, nheads, width),
        out_type=jax.ShapeDtypeStruct((2, NPAD, width), jnp.float32),
        mesh=_mesh(),
        compiler_params=pltpu.CompilerParams(use_tc_tiling_on_sc=False),
        scratch_types=bufs + [
            pltpu.VMEM((16,), jnp.float32),
            pltpu.VMEM_SHARED((NPAD, width), jnp.float32),
            pltpu.SemaphoreType.DMA,
            pltpu.SemaphoreType.DMA,
        ],
    )
    return f(ab, ba, src, dst, gvec, table, zw)


def kernel(x, edge_index, W1, att_src1, att_dst1, b1, W2, att_src2, att_dst2, b2):
    f32 = jnp.float32
    xpad = jnp.zeros((NPAD, D), f32).at[:N].set(x)
    loop = jnp.arange(N, dtype=jnp.int32)
    padi = jnp.full((EP - E - N,), DUMMY, dtype=jnp.int32)
    src = jnp.concatenate([edge_index[0].astype(jnp.int32), loop, padi])
    dst = jnp.concatenate([edge_index[1].astype(jnp.int32), loop, padi])

    eye8 = jnp.eye(HEADS, dtype=f32)
    As = (att_src1[:, :, None] * eye8[:, None, :]).reshape(D, HEADS)
    Ad = (att_dst1[:, :, None] * eye8[:, None, :]).reshape(D, HEADS)
    zero8 = jnp.zeros((8, 16), f32)
    cab = jnp.concatenate([jnp.concatenate([As, Ad], axis=1), zero8], axis=0)
    cba = jnp.concatenate([jnp.concatenate([Ad, As], axis=1), zero8], axis=0)

    W1e = jnp.concatenate([W1, jnp.zeros((D, 8), f32)], axis=1)
    ones1 = jnp.zeros((1, W1W), f32).at[0, D:].set(1.0)
    W2e = jnp.zeros((D, W2W), f32).at[:, :OUT].set(W2)
    ones2 = jnp.zeros((1, W2W), f32).at[0, OUT:].set(1.0)
    t2 = jnp.zeros((W2W, 16), f32).at[:OUT, 0].set(att_src2[0]).at[:OUT, 1].set(att_dst2[0])
    t2b = jnp.zeros((W2W, 16), f32).at[:OUT, 0].set(att_dst2[0]).at[:OUT, 1].set(att_src2[0])
    b1r = b1.reshape(1, D)
    b2p = jnp.zeros((1, W2W), f32).at[0, :OUT].set(b2)

    rexp = jnp.zeros((8, D), f32)
    for h in range(HEADS):
        rexp = rexp.at[h, h * 16:(h + 1) * 16].set(1.0)
    r2 = jnp.zeros((8, W2W), f32).at[0, :].set(1.0)

    zw1 = jnp.zeros((NPAD, W1W), f32)
    zw2 = jnp.zeros((NPAD, W2W), f32)

    h, ab1, ba1, gm1 = _tc1(xpad, W1e, ones1, cab, cba)
    g8 = gm1[0, :8] + gm1[0, 8:]
    g1vec = jnp.concatenate([g8, g8])
    acc1 = _sc_edge(HEADS, W1W, ab1, ba1, src, dst, g1vec, h, zw1)

    g2, ab2, ba2, gm2 = _tc2(acc1[0], acc1[1], rexp, b1r, W2e, ones2, t2, t2b)
    g2vec = jnp.full((16,), gm2[0, 0] + gm2[0, 1], f32)
    acc2 = _sc_edge(1, W2W, ab2, ba2, src, dst, g2vec, g2, zw2)

    out = _tc3(acc2[0], acc2[1], r2, b2p)
    return out[:N, :OUT]


# speedup vs baseline: 1.2375x; 1.2375x over previous
import functools

import jax
import jax.numpy as jnp
from jax import lax
from jax.experimental import pallas as pl
from jax.experimental.pallas import tpu as pltpu
from jax.experimental.pallas import tpu_sc as plsc

N = 10000
D = 128
HEADS = 8
C1 = 16
OUT = 40
E = 320000

NPAD = 10016
DUMMY = N
W1W = D + 8
W2W = 48
NW = 32
CH = 82
CHUNK = 128
NPW = CH * CHUNK
EP = NW * NPW
ROWS_PER_TILE = NPAD // 16
TCBLK = 2504


def _mesh():
    return plsc.VectorSubcoreMesh(core_axis_name="c", subcore_axis_name="s")


def _tc1_body(x_ref, w_ref, ones_ref, cab_ref, cba_ref,
              h_ref, ab_ref, ba_ref, g_ref):
    xb = x_ref[...]
    hb = jnp.dot(xb, w_ref[...], preferred_element_type=jnp.float32)
    hb = hb + ones_ref[...]
    h_ref[...] = hb
    ab = jnp.dot(hb, cab_ref[...], preferred_element_type=jnp.float32)
    ba = jnp.dot(hb, cba_ref[...], preferred_element_type=jnp.float32)
    ab_ref[...] = ab
    ba_ref[...] = ba
    m = jnp.max(ab, axis=0, keepdims=True)
    i = pl.program_id(0)

    @pl.when(i == 0)
    def _():
        g_ref[...] = m

    @pl.when(i > 0)
    def _():
        g_ref[...] = jnp.maximum(g_ref[...], m)


def _tc1(xpad, W1e, ones1, cab, cba):
    grid = NPAD // TCBLK
    return pl.pallas_call(
        _tc1_body,
        grid=(grid,),
        in_specs=[
            pl.BlockSpec((TCBLK, D), lambda i: (i, 0)),
            pl.BlockSpec((D, W1W), lambda i: (0, 0)),
            pl.BlockSpec((1, W1W), lambda i: (0, 0)),
            pl.BlockSpec((W1W, 16), lambda i: (0, 0)),
            pl.BlockSpec((W1W, 16), lambda i: (0, 0)),
        ],
        out_specs=[
            pl.BlockSpec((TCBLK, W1W), lambda i: (i, 0)),
            pl.BlockSpec((TCBLK, 16), lambda i: (i, 0)),
            pl.BlockSpec((TCBLK, 16), lambda i: (i, 0)),
            pl.BlockSpec((1, 16), lambda i: (0, 0)),
        ],
        out_shape=[
            jax.ShapeDtypeStruct((NPAD, W1W), jnp.float32),
            jax.ShapeDtypeStruct((NPAD, 16), jnp.float32),
            jax.ShapeDtypeStruct((NPAD, 16), jnp.float32),
            jax.ShapeDtypeStruct((1, 16), jnp.float32),
        ],
    )(xpad, W1e, ones1, cab, cba)


def _tc2_body(a_ref, b_ref, rexp_ref, bias_ref, w2_ref, ones_ref,
              t2_ref, t2b_ref, g2_ref, ab_ref, ba_ref, gm_ref):
    acc = a_ref[...] + b_ref[...]
    msg = acc[:, :D]
    den = jnp.dot(acc[:, D:], rexp_ref[...], preferred_element_type=jnp.float32)
    hb = msg / (den + 1e-16) + bias_ref[...]
    hb = jnp.where(hb > 0, hb, jnp.exp(hb) - 1.0)
    g2 = jnp.dot(hb, w2_ref[...], preferred_element_type=jnp.float32)
    g2 = g2 + ones_ref[...]
    g2_ref[...] = g2
    ab = jnp.dot(g2, t2_ref[...], preferred_element_type=jnp.float32)
    ba = jnp.dot(g2, t2b_ref[...], preferred_element_type=jnp.float32)
    ab_ref[...] = ab
    ba_ref[...] = ba
    m = jnp.max(ab, axis=0, keepdims=True)
    i = pl.program_id(0)

    @pl.when(i == 0)
    def _():
        gm_ref[...] = m

    @pl.when(i > 0)
    def _():
        gm_ref[...] = jnp.maximum(gm_ref[...], m)


def _tc2(accA, accB, rexp, b1r, W2e, ones2, t2, t2b):
    grid = NPAD // TCBLK
    return pl.pallas_call(
        _tc2_body,
        grid=(grid,),
        in_specs=[
            pl.BlockSpec((TCBLK, W1W), lambda i: (i, 0)),
            pl.BlockSpec((TCBLK, W1W), lambda i: (i, 0)),
            pl.BlockSpec((8, D), lambda i: (0, 0)),
            pl.BlockSpec((1, D), lambda i: (0, 0)),
            pl.BlockSpec((D, W2W), lambda i: (0, 0)),
            pl.BlockSpec((1, W2W), lambda i: (0, 0)),
            pl.BlockSpec((W2W, 16), lambda i: (0, 0)),
            pl.BlockSpec((W2W, 16), lambda i: (0, 0)),
        ],
        out_specs=[
            pl.BlockSpec((TCBLK, W2W), lambda i: (i, 0)),
            pl.BlockSpec((TCBLK, 16), lambda i: (i, 0)),
            pl.BlockSpec((TCBLK, 16), lambda i: (i, 0)),
            pl.BlockSpec((1, 16), lambda i: (0, 0)),
        ],
        out_shape=[
            jax.ShapeDtypeStruct((NPAD, W2W), jnp.float32),
            jax.ShapeDtypeStruct((NPAD, 16), jnp.float32),
            jax.ShapeDtypeStruct((NPAD, 16), jnp.float32),
            jax.ShapeDtypeStruct((1, 16), jnp.float32),
        ],
    )(accA, accB, rexp, b1r, W2e, ones2, t2, t2b)


def _tc3_body(a_ref, b_ref, r2_ref, bias_ref, o_ref):
    acc = a_ref[...] + b_ref[...]
    den = jnp.dot(acc[:, OUT:], r2_ref[...], preferred_element_type=jnp.float32)
    o_ref[...] = acc / (den + 1e-16) + bias_ref[...]


def _tc3(accA, accB, r2, b2p):
    grid = NPAD // TCBLK
    return pl.pallas_call(
        _tc3_body,
        grid=(grid,),
        in_specs=[
            pl.BlockSpec((TCBLK, W2W), lambda i: (i, 0)),
            pl.BlockSpec((TCBLK, W2W), lambda i: (i, 0)),
            pl.BlockSpec((8, W2W), lambda i: (0, 0)),
            pl.BlockSpec((1, W2W), lambda i: (0, 0)),
        ],
        out_specs=pl.BlockSpec((TCBLK, W2W), lambda i: (i, 0)),
        out_shape=jax.ShapeDtypeStruct((NPAD, W2W), jnp.float32),
    )(accA, accB, r2, b2p)


def _sc_edge_body(nheads, width, ab_hbm, ba_hbm, src_hbm, dst_hbm, g_hbm,
                  tab_hbm, zw_hbm, acc_hbm,
                  sidxA, didxA, raA, rbA, rowA,
                  sidxB, didxB, raB, rbB, rowB,
                  gv, acc, semA, semB):
    cid = lax.axis_index("c")
    sid = lax.axis_index("s")
    wid = sid * 2 + cid
    row0 = sid * ROWS_PER_TILE
    pltpu.sync_copy(zw_hbm.at[pl.ds(row0, ROWS_PER_TILE)],
                    acc.at[pl.ds(row0, ROWS_PER_TILE)])
    pltpu.sync_copy(g_hbm, gv)
    plsc.subcore_barrier()
    g = gv[...]
    lane = lax.iota(jnp.int32, 16)
    shift_idx = jnp.where(lane >= 8, lane - 8, 0)

    def issue(c, sidx, didx, ra, rb, row, sem):
        base = pl.multiple_of((wid * CH + jnp.minimum(c, CH - 1)) * CHUNK, CHUNK)
        pltpu.sync_copy(src_hbm.at[pl.ds(base, CHUNK)], sidx)
        pltpu.sync_copy(dst_hbm.at[pl.ds(base, CHUNK)], didx)
        cp1 = pltpu.async_copy(ab_hbm.at[sidx], ra, sem)
        cp2 = pltpu.async_copy(ba_hbm.at[didx], rb, sem)
        cp3 = pltpu.async_copy(tab_hbm.at[sidx], row, sem)
        return cp1, cp2, cp3

    def compute(didx, ra, rb, row, cps):
        for cp in cps:
            cp.wait()

        def inner(k, c2):
            e = ra[k, :] + rb[k, :]
            e = jnp.maximum(e, 0.2 * e)
            p = jnp.exp(e - g)
            if nheads == 1:
                p0 = p[0]
                for j in range(width // 16):
                    row[k, j * 16:(j + 1) * 16] = row[k, j * 16:(j + 1) * 16] * p0
            else:
                for h in range(nheads):
                    row[k, h * 16:(h + 1) * 16] = (
                        row[k, h * 16:(h + 1) * 16] * p[h])
                ps = jnp.where(lane >= 8, jnp.take(p, shift_idx), 1.0)
                row[k, width - 16:width] = row[k, width - 16:width] * ps
            return c2

        lax.fori_loop(0, CHUNK, inner, 0, unroll=4)
        pltpu.sync_copy(row, acc.at[didx], add=True)
        return None

    cpsA = issue(0, sidxA, didxA, raA, rbA, rowA, semA)
    cpsB = issue(1, sidxB, didxB, raB, rbB, rowB, semB)

    def outer(j, carry):
        compute(didxA, raA, rbA, rowA, cpsA)
        issue(2 * j + 2, sidxA, didxA, raA, rbA, rowA, semA)
        compute(didxB, raB, rbB, rowB, cpsB)
        issue(2 * j + 3, sidxB, didxB, raB, rbB, rowB, semB)
        return carry

    lax.fori_loop(0, CH // 2, outer, 0)
    for cps in (cpsA, cpsB):
        for cp in cps:
            cp.wait()
    plsc.subcore_barrier()
    pltpu.sync_copy(acc.at[pl.ds(row0, ROWS_PER_TILE)],
                    acc_hbm.at[cid].at[pl.ds(row0, ROWS_PER_TILE)])


def _sc_edge(nheads, width, ab, ba, src, dst, gvec, table, zw):
    bufs = []
    for _ in range(2):
        bufs += [
            pltpu.VMEM((CHUNK,), jnp.int32),
            pltpu.VMEM((CHUNK,), jnp.int32),
            pltpu.VMEM((CHUNK, 16), jnp.float32),
            pltpu.VMEM((CHUNK, 16), jnp.float32),
            pltpu.VMEM((CHUNK, width), jnp.float32),
        ]
    f = pl.kernel(
        functools.partial(_sc_edge_body, nheads, width),
        out_type=jax.ShapeDtypeStruct((2, NPAD, width), jnp.float32),
        mesh=_mesh(),
        compiler_params=pltpu.CompilerParams(use_tc_tiling_on_sc=False),
        scratch_types=bufs + [
            pltpu.VMEM((16,), jnp.float32),
            pltpu.VMEM_SHARED((NPAD, width), jnp.float32),
            pltpu.SemaphoreType.DMA,
            pltpu.SemaphoreType.DMA,
        ],
    )
    return f(ab, ba, src, dst, gvec, table, zw)


def kernel(x, edge_index, W1, att_src1, att_dst1, b1, W2, att_src2, att_dst2, b2):
    f32 = jnp.float32
    xpad = jnp.zeros((NPAD, D), f32).at[:N].set(x)
    loop = jnp.arange(N, dtype=jnp.int32)
    padi = jnp.full((EP - E - N,), DUMMY, dtype=jnp.int32)
    src = jnp.concatenate([edge_index[0].astype(jnp.int32), loop, padi])
    dst = jnp.concatenate([edge_index[1].astype(jnp.int32), loop, padi])

    eye8 = jnp.eye(HEADS, dtype=f32)
    As = (att_src1[:, :, None] * eye8[:, None, :]).reshape(D, HEADS)
    Ad = (att_dst1[:, :, None] * eye8[:, None, :]).reshape(D, HEADS)
    zero8 = jnp.zeros((8, 16), f32)
    cab = jnp.concatenate([jnp.concatenate([As, Ad], axis=1), zero8], axis=0)
    cba = jnp.concatenate([jnp.concatenate([Ad, As], axis=1), zero8], axis=0)

    W1e = jnp.concatenate([W1, jnp.zeros((D, 8), f32)], axis=1)
    ones1 = jnp.zeros((1, W1W), f32).at[0, D:].set(1.0)
    W2e = jnp.zeros((D, W2W), f32).at[:, :OUT].set(W2)
    ones2 = jnp.zeros((1, W2W), f32).at[0, OUT:].set(1.0)
    t2 = jnp.zeros((W2W, 16), f32).at[:OUT, 0].set(att_src2[0]).at[:OUT, 1].set(att_dst2[0])
    t2b = jnp.zeros((W2W, 16), f32).at[:OUT, 0].set(att_dst2[0]).at[:OUT, 1].set(att_src2[0])
    b1r = b1.reshape(1, D)
    b2p = jnp.zeros((1, W2W), f32).at[0, :OUT].set(b2)

    rexp = jnp.zeros((8, D), f32)
    for h in range(HEADS):
        rexp = rexp.at[h, h * 16:(h + 1) * 16].set(1.0)
    r2 = jnp.zeros((8, W2W), f32).at[0, :].set(1.0)

    zw1 = jnp.zeros((NPAD, W1W), f32)
    zw2 = jnp.zeros((NPAD, W2W), f32)

    h, ab1, ba1, gm1 = _tc1(xpad, W1e, ones1, cab, cba)
    g8 = gm1[0, :8] + gm1[0, 8:]
    g1vec = jnp.concatenate([g8, g8])
    acc1 = _sc_edge(HEADS, W1W, ab1, ba1, src, dst, g1vec, h, zw1)

    g2, ab2, ba2, gm2 = _tc2(acc1[0], acc1[1], rexp, b1r, W2e, ones2, t2, t2b)
    g2vec = jnp.full((16,), gm2[0, 0] + gm2[0, 1], f32)
    acc2 = _sc_edge(1, W2W, ab2, ba2, src, dst, g2vec, g2, zw2)

    out = _tc3(acc2[0], acc2[1], r2, b2p)
    return out[:N, :OUT]


# speedup vs baseline: 1.4978x; 1.2103x over previous
import functools

import jax
import jax.numpy as jnp
from jax import lax
from jax.experimental import pallas as pl
from jax.experimental.pallas import tpu as pltpu
from jax.experimental.pallas import tpu_sc as plsc

N = 10000
D = 128
HEADS = 8
C1 = 16
OUT = 40
E = 320000

NPAD = 10016
DUMMY = N
W1W = D + 8
W2W = 48
NW = 32
CH = 82
CHUNK = 128
NPW = CH * CHUNK
EP = NW * NPW
ROWS_PER_TILE = NPAD // 16
TCBLK = 2504


def _mesh():
    return plsc.VectorSubcoreMesh(core_axis_name="c", subcore_axis_name="s")


def _tc1_body(x_ref, w_ref, ones_ref, cab_ref, cba_ref,
              h_ref, ab_ref, ba_ref, g_ref):
    xb = x_ref[...]
    hb = jnp.dot(xb, w_ref[...], preferred_element_type=jnp.float32)
    hb = hb + ones_ref[...]
    h_ref[...] = hb
    ab = jnp.dot(hb, cab_ref[...], preferred_element_type=jnp.float32)
    ba = jnp.dot(hb, cba_ref[...], preferred_element_type=jnp.float32)
    ab_ref[...] = ab
    ba_ref[...] = ba
    m = jnp.max(ab, axis=0, keepdims=True)
    i = pl.program_id(0)

    @pl.when(i == 0)
    def _():
        g_ref[...] = m

    @pl.when(i > 0)
    def _():
        g_ref[...] = jnp.maximum(g_ref[...], m)


def _tc1(xpad, W1e, ones1, cab, cba):
    grid = NPAD // TCBLK
    return pl.pallas_call(
        _tc1_body,
        grid=(grid,),
        in_specs=[
            pl.BlockSpec((TCBLK, D), lambda i: (i, 0)),
            pl.BlockSpec((D, W1W), lambda i: (0, 0)),
            pl.BlockSpec((1, W1W), lambda i: (0, 0)),
            pl.BlockSpec((W1W, 16), lambda i: (0, 0)),
            pl.BlockSpec((W1W, 16), lambda i: (0, 0)),
        ],
        out_specs=[
            pl.BlockSpec((TCBLK, W1W), lambda i: (i, 0)),
            pl.BlockSpec((TCBLK, 16), lambda i: (i, 0)),
            pl.BlockSpec((TCBLK, 16), lambda i: (i, 0)),
            pl.BlockSpec((1, 16), lambda i: (0, 0)),
        ],
        out_shape=[
            jax.ShapeDtypeStruct((NPAD, W1W), jnp.float32),
            jax.ShapeDtypeStruct((NPAD, 16), jnp.float32),
            jax.ShapeDtypeStruct((NPAD, 16), jnp.float32),
            jax.ShapeDtypeStruct((1, 16), jnp.float32),
        ],
    )(xpad, W1e, ones1, cab, cba)


def _tc2_body(a_ref, b_ref, rexp_ref, bias_ref, w2_ref, ones_ref,
              t2_ref, t2b_ref, g2_ref, ab_ref, ba_ref, gm_ref):
    acc = a_ref[...] + b_ref[...]
    msg = acc[:, :D]
    den = jnp.dot(acc[:, D:], rexp_ref[...], preferred_element_type=jnp.float32)
    hb = msg / (den + 1e-16) + bias_ref[...]
    hb = jnp.where(hb > 0, hb, jnp.exp(hb) - 1.0)
    g2 = jnp.dot(hb, w2_ref[...], preferred_element_type=jnp.float32)
    g2 = g2 + ones_ref[...]
    g2_ref[...] = g2
    ab = jnp.dot(g2, t2_ref[...], preferred_element_type=jnp.float32)
    ba = jnp.dot(g2, t2b_ref[...], preferred_element_type=jnp.float32)
    ab_ref[...] = ab
    ba_ref[...] = ba
    m = jnp.max(ab, axis=0, keepdims=True)
    i = pl.program_id(0)

    @pl.when(i == 0)
    def _():
        gm_ref[...] = m

    @pl.when(i > 0)
    def _():
        gm_ref[...] = jnp.maximum(gm_ref[...], m)


def _tc2(accA, accB, rexp, b1r, W2e, ones2, t2, t2b):
    grid = NPAD // TCBLK
    return pl.pallas_call(
        _tc2_body,
        grid=(grid,),
        in_specs=[
            pl.BlockSpec((TCBLK, W1W), lambda i: (i, 0)),
            pl.BlockSpec((TCBLK, W1W), lambda i: (i, 0)),
            pl.BlockSpec((8, D), lambda i: (0, 0)),
            pl.BlockSpec((1, D), lambda i: (0, 0)),
            pl.BlockSpec((D, W2W), lambda i: (0, 0)),
            pl.BlockSpec((1, W2W), lambda i: (0, 0)),
            pl.BlockSpec((W2W, 16), lambda i: (0, 0)),
            pl.BlockSpec((W2W, 16), lambda i: (0, 0)),
        ],
        out_specs=[
            pl.BlockSpec((TCBLK, W2W), lambda i: (i, 0)),
            pl.BlockSpec((TCBLK, 16), lambda i: (i, 0)),
            pl.BlockSpec((TCBLK, 16), lambda i: (i, 0)),
            pl.BlockSpec((1, 16), lambda i: (0, 0)),
        ],
        out_shape=[
            jax.ShapeDtypeStruct((NPAD, W2W), jnp.float32),
            jax.ShapeDtypeStruct((NPAD, 16), jnp.float32),
            jax.ShapeDtypeStruct((NPAD, 16), jnp.float32),
            jax.ShapeDtypeStruct((1, 16), jnp.float32),
        ],
    )(accA, accB, rexp, b1r, W2e, ones2, t2, t2b)


def _tc3_body(a_ref, b_ref, r2_ref, bias_ref, o_ref):
    acc = a_ref[...] + b_ref[...]
    den = jnp.dot(acc[:, OUT:], r2_ref[...], preferred_element_type=jnp.float32)
    o_ref[...] = acc / (den + 1e-16) + bias_ref[...]


def _tc3(accA, accB, r2, b2p):
    grid = NPAD // TCBLK
    return pl.pallas_call(
        _tc3_body,
        grid=(grid,),
        in_specs=[
            pl.BlockSpec((TCBLK, W2W), lambda i: (i, 0)),
            pl.BlockSpec((TCBLK, W2W), lambda i: (i, 0)),
            pl.BlockSpec((8, W2W), lambda i: (0, 0)),
            pl.BlockSpec((1, W2W), lambda i: (0, 0)),
        ],
        out_specs=pl.BlockSpec((TCBLK, W2W), lambda i: (i, 0)),
        out_shape=jax.ShapeDtypeStruct((NPAD, W2W), jnp.float32),
    )(accA, accB, r2, b2p)


def _sc_edge_body(nheads, width, ab_hbm, ba_hbm, src_hbm, dst_hbm, g_hbm,
                  tab_hbm, zw_hbm, acc_hbm,
                  sidxA, didxA, raA, rbA, rowA,
                  sidxB, didxB, raB, rbB, rowB,
                  gv, acc, semA, semB):
    cid = lax.axis_index("c")
    sid = lax.axis_index("s")
    wid = sid * 2 + cid
    row0 = sid * ROWS_PER_TILE
    pltpu.sync_copy(zw_hbm.at[pl.ds(row0, ROWS_PER_TILE)],
                    acc.at[pl.ds(row0, ROWS_PER_TILE)])
    pltpu.sync_copy(g_hbm, gv)
    plsc.subcore_barrier()
    g = gv[...]
    lane = lax.iota(jnp.int32, 16)
    shift_idx = jnp.where(lane >= 8, lane - 8, 0)

    def issue(c, sidx, didx, ra, rb, row, sem):
        base = pl.multiple_of((wid * CH + jnp.minimum(c, CH - 1)) * CHUNK, CHUNK)
        pltpu.sync_copy(src_hbm.at[pl.ds(base, CHUNK)], sidx)
        pltpu.sync_copy(dst_hbm.at[pl.ds(base, CHUNK)], didx)
        cp1 = pltpu.async_copy(ab_hbm.at[sidx], ra, sem)
        cp2 = pltpu.async_copy(ba_hbm.at[didx], rb, sem)
        cp3 = pltpu.async_copy(tab_hbm.at[sidx], row, sem)
        return cp1, cp2, cp3

    def compute(didx, ra, rb, row, cps):
        for cp in cps:
            cp.wait()

        @plsc.parallel_loop(0, CHUNK, step=1, unroll=4)
        def inner(k):
            e = ra[k, :] + rb[k, :]
            e = jnp.maximum(e, 0.2 * e)
            p = jnp.exp(e - g)
            if nheads == 1:
                p0 = p[0]
                for j in range(width // 16):
                    row[k, j * 16:(j + 1) * 16] = row[k, j * 16:(j + 1) * 16] * p0
            else:
                for h in range(nheads):
                    row[k, h * 16:(h + 1) * 16] = (
                        row[k, h * 16:(h + 1) * 16] * p[h])
                ps = jnp.where(lane >= 8, jnp.take(p, shift_idx), 1.0)
                row[k, width - 16:width] = row[k, width - 16:width] * ps
        pltpu.sync_copy(row, acc.at[didx], add=True)
        return None

    cpsA = issue(0, sidxA, didxA, raA, rbA, rowA, semA)
    cpsB = issue(1, sidxB, didxB, raB, rbB, rowB, semB)

    def outer(j, carry):
        compute(didxA, raA, rbA, rowA, cpsA)
        issue(2 * j + 2, sidxA, didxA, raA, rbA, rowA, semA)
        compute(didxB, raB, rbB, rowB, cpsB)
        issue(2 * j + 3, sidxB, didxB, raB, rbB, rowB, semB)
        return carry

    lax.fori_loop(0, CH // 2, outer, 0)
    for cps in (cpsA, cpsB):
        for cp in cps:
            cp.wait()
    plsc.subcore_barrier()
    pltpu.sync_copy(acc.at[pl.ds(row0, ROWS_PER_TILE)],
                    acc_hbm.at[cid].at[pl.ds(row0, ROWS_PER_TILE)])


def _sc_edge(nheads, width, ab, ba, src, dst, gvec, table, zw):
    bufs = []
    for _ in range(2):
        bufs += [
            pltpu.VMEM((CHUNK,), jnp.int32),
            pltpu.VMEM((CHUNK,), jnp.int32),
            pltpu.VMEM((CHUNK, 16), jnp.float32),
            pltpu.VMEM((CHUNK, 16), jnp.float32),
            pltpu.VMEM((CHUNK, width), jnp.float32),
        ]
    f = pl.kernel(
        functools.partial(_sc_edge_body, nheads, width),
        out_type=jax.ShapeDtypeStruct((2, NPAD, width), jnp.float32),
        mesh=_mesh(),
        compiler_params=pltpu.CompilerParams(use_tc_tiling_on_sc=False),
        scratch_types=bufs + [
            pltpu.VMEM((16,), jnp.float32),
            pltpu.VMEM_SHARED((NPAD, width), jnp.float32),
            pltpu.SemaphoreType.DMA,
            pltpu.SemaphoreType.DMA,
        ],
    )
    return f(ab, ba, src, dst, gvec, table, zw)


def kernel(x, edge_index, W1, att_src1, att_dst1, b1, W2, att_src2, att_dst2, b2):
    f32 = jnp.float32
    xpad = jnp.zeros((NPAD, D), f32).at[:N].set(x)
    loop = jnp.arange(N, dtype=jnp.int32)
    padi = jnp.full((EP - E - N,), DUMMY, dtype=jnp.int32)
    src = jnp.concatenate([edge_index[0].astype(jnp.int32), loop, padi])
    dst = jnp.concatenate([edge_index[1].astype(jnp.int32), loop, padi])

    eye8 = jnp.eye(HEADS, dtype=f32)
    As = (att_src1[:, :, None] * eye8[:, None, :]).reshape(D, HEADS)
    Ad = (att_dst1[:, :, None] * eye8[:, None, :]).reshape(D, HEADS)
    zero8 = jnp.zeros((8, 16), f32)
    cab = jnp.concatenate([jnp.concatenate([As, Ad], axis=1), zero8], axis=0)
    cba = jnp.concatenate([jnp.concatenate([Ad, As], axis=1), zero8], axis=0)

    W1e = jnp.concatenate([W1, jnp.zeros((D, 8), f32)], axis=1)
    ones1 = jnp.zeros((1, W1W), f32).at[0, D:].set(1.0)
    W2e = jnp.zeros((D, W2W), f32).at[:, :OUT].set(W2)
    ones2 = jnp.zeros((1, W2W), f32).at[0, OUT:].set(1.0)
    t2 = jnp.zeros((W2W, 16), f32).at[:OUT, 0].set(att_src2[0]).at[:OUT, 1].set(att_dst2[0])
    t2b = jnp.zeros((W2W, 16), f32).at[:OUT, 0].set(att_dst2[0]).at[:OUT, 1].set(att_src2[0])
    b1r = b1.reshape(1, D)
    b2p = jnp.zeros((1, W2W), f32).at[0, :OUT].set(b2)

    rexp = jnp.zeros((8, D), f32)
    for h in range(HEADS):
        rexp = rexp.at[h, h * 16:(h + 1) * 16].set(1.0)
    r2 = jnp.zeros((8, W2W), f32).at[0, :].set(1.0)

    zw1 = jnp.zeros((NPAD, W1W), f32)
    zw2 = jnp.zeros((NPAD, W2W), f32)

    h, ab1, ba1, gm1 = _tc1(xpad, W1e, ones1, cab, cba)
    g8 = gm1[0, :8] + gm1[0, 8:]
    g1vec = jnp.concatenate([g8, g8])
    acc1 = _sc_edge(HEADS, W1W, ab1, ba1, src, dst, g1vec, h, zw1)

    g2, ab2, ba2, gm2 = _tc2(acc1[0], acc1[1], rexp, b1r, W2e, ones2, t2, t2b)
    g2vec = jnp.full((16,), gm2[0, 0] + gm2[0, 1], f32)
    acc2 = _sc_edge(1, W2W, ab2, ba2, src, dst, g2vec, g2, zw2)

    out = _tc3(acc2[0], acc2[1], r2, b2p)
    return out[:N, :OUT]
